# baseline (speedup 1.0000x reference)
import functools

import jax
import jax.numpy as jnp
from jax.experimental import pallas as pl
from jax.experimental.pallas import tpu as pltpu

_BLOCK_M = 512


def _router_block(x_ref, w_ref, o_ref):
    o_ref[...] = jax.lax.dot_general(
        x_ref[...],
        w_ref[...],
        dimension_numbers=(((1,), (1,)), ((), ())),
        preferred_element_type=jnp.float32,
    )


@functools.partial(jax.jit, static_argnames=())
def kernel(x, W):
    B, T, D = x.shape
    E = W.shape[0]
    M = B * T
    x2 = x.reshape(M, D)
    block_m = _BLOCK_M
    grid = (M // block_m,)
    out = pl.pallas_call(
        _router_block,
        grid=grid,
        in_specs=[
            pl.BlockSpec((block_m, D), lambda i: (i, 0)),
            pl.BlockSpec((E, D), lambda i: (0, 0)),
        ],
        out_specs=pl.BlockSpec((block_m, E), lambda i: (i, 0)),
        out_shape=jax.ShapeDtypeStruct((M, E), jnp.float32),
        compiler_params=pltpu.CompilerParams(
            dimension_semantics=("arbitrary",),
        ),
    )(x2, W)
    return out.reshape(B, T, E)


# speedup vs baseline: 1.1268x; 1.1268x over previous
import functools

import jax
import jax.numpy as jnp
from jax.experimental import pallas as pl
from jax.experimental.pallas import tpu as pltpu

_BLOCK_M = 2048


def _router_block(x_ref, w_ref, o_ref):
    o_ref[...] = jax.lax.dot_general(
        x_ref[...],
        w_ref[...],
        dimension_numbers=(((1,), (1,)), ((), ())),
        preferred_element_type=jnp.float32,
    )


@functools.partial(jax.jit, static_argnames=())
def kernel(x, W):
    B, T, D = x.shape
    E = W.shape[0]
    M = B * T
    x2 = x.reshape(M, D)
    block_m = _BLOCK_M
    grid = (M // block_m,)
    out = pl.pallas_call(
        _router_block,
        grid=grid,
        in_specs=[
            pl.BlockSpec((block_m, D), lambda i: (i, 0)),
            pl.BlockSpec((E, D), lambda i: (0, 0)),
        ],
        out_specs=pl.BlockSpec((block_m, E), lambda i: (i, 0)),
        out_shape=jax.ShapeDtypeStruct((M, E), jnp.float32),
        compiler_params=pltpu.CompilerParams(
            dimension_semantics=("arbitrary",),
        ),
    )(x2, W)
    return out.reshape(B, T, E)


# speedup vs baseline: 1.1273x; 1.0004x over previous
import functools

import jax
import jax.numpy as jnp
from jax.experimental import pallas as pl
from jax.experimental.pallas import tpu as pltpu

_BLOCK_M = 1024
_NSTREAM = 2


def _router_block(xa_ref, xb_ref, w_ref, o_ref):
    dn = (((1,), (1,)), ((), ()))
    o_ref[pl.ds(0, _BLOCK_M), :] = jax.lax.dot_general(
        xa_ref[...], w_ref[...], dimension_numbers=dn,
        preferred_element_type=jnp.float32)
    o_ref[pl.ds(_BLOCK_M, _BLOCK_M), :] = jax.lax.dot_general(
        xb_ref[...], w_ref[...], dimension_numbers=dn,
        preferred_element_type=jnp.float32)


@functools.partial(jax.jit, static_argnames=())
def kernel(x, W):
    B, T, D = x.shape
    E = W.shape[0]
    M = B * T
    x2 = x.reshape(M, D)
    super_m = _BLOCK_M * _NSTREAM
    grid = (M // super_m,)
    out = pl.pallas_call(
        _router_block,
        grid=grid,
        in_specs=[
            pl.BlockSpec((_BLOCK_M, D), lambda i: (2 * i, 0)),
            pl.BlockSpec((_BLOCK_M, D), lambda i: (2 * i + 1, 0)),
            pl.BlockSpec((E, D), lambda i: (0, 0)),
        ],
        out_specs=pl.BlockSpec((super_m, E), lambda i: (i, 0)),
        out_shape=jax.ShapeDtypeStruct((M, E), jnp.float32),
        compiler_params=pltpu.CompilerParams(
            dimension_semantics=("arbitrary",),
        ),
    )(x2, x2, W)
    return out.reshape(B, T, E)
